# baseline (speedup 1.0000x reference)
import functools

import jax
import jax.numpy as jnp
from jax import lax
from jax.experimental import pallas as pl
from jax.experimental.pallas import tpu as pltpu
from jax.experimental.pallas import tpu_sc as plsc

N = 10000
E = 320000
D = 128
DE = 16
DIM = 100
G = N // DIM
H1 = 512
OUT = 128

NC, NS = 2, 16
NW = NC * NS
K = 64
NB = 160
NCH = 8
CH = NB // NCH
E_PAD = NW * NB * K
N_ACC = 10240
ZERO_PT = N_ACC // NS


def _gather_sweep_body(h_hbm, src_hbm, dst_hbm, z_hbm, out_x,
                       src_v, dst_v, rb0, rb1, rb2, rb3, acc,
                       zsem, sem0, sem1, sem2, sem3):
    c = lax.axis_index("c")
    s = lax.axis_index("s")
    wid = c * NS + s

    zbase = s * ZERO_PT
    pltpu.async_copy(z_hbm, acc.at[pl.ds(zbase, ZERO_PT)], zsem)
    pltpu.make_async_copy(z_hbm, acc.at[pl.ds(zbase, ZERO_PT)], zsem).wait()

    plsc.subcore_barrier()

    def chunk(ci, carry):
        pltpu.sync_copy(src_hbm.at[wid, ci], src_v)
        pltpu.sync_copy(dst_hbm.at[wid, ci], dst_v)
        pltpu.async_copy(h_hbm.at[src_v.at[0]], rb0, sem0)
        pltpu.async_copy(h_hbm.at[src_v.at[1]], rb1, sem1)
        pltpu.async_copy(h_hbm.at[src_v.at[2]], rb2, sem2)
        pltpu.async_copy(h_hbm.at[src_v.at[3]], rb3, sem3)

        def step(rb, sem, j):
            pltpu.make_async_copy(h_hbm.at[src_v.at[j]], rb, sem).wait()
            pltpu.sync_copy(rb, acc.at[dst_v.at[j]], add=True)

            @pl.when(j + 4 < CH)
            def _():
                pltpu.async_copy(h_hbm.at[src_v.at[j + 4]], rb, sem)

        def quad(i, c2):
            step(rb0, sem0, 4 * i)
            step(rb1, sem1, 4 * i + 1)
            step(rb2, sem2, 4 * i + 2)
            step(rb3, sem3, 4 * i + 3)
            return c2

        lax.fori_loop(0, CH // 4, quad, 0)
        return carry

    lax.fori_loop(0, NCH, chunk, 0)

    plsc.subcore_barrier()

    pltpu.sync_copy(acc.at[pl.ds(zbase, ZERO_PT)],
                    out_x.at[c, pl.ds(zbase, ZERO_PT)])


_sweep = pl.kernel(
    _gather_sweep_body,
    out_type=(jax.ShapeDtypeStruct((NC, N_ACC, D), jnp.float32),),
    mesh=plsc.VectorSubcoreMesh(core_axis_name="c", subcore_axis_name="s"),
    scratch_types=[
        pltpu.VMEM((CH, K), jnp.int32),
        pltpu.VMEM((CH, K), jnp.int32),
        pltpu.VMEM((K, D), jnp.float32),
        pltpu.VMEM((K, D), jnp.float32),
        pltpu.VMEM((K, D), jnp.float32),
        pltpu.VMEM((K, D), jnp.float32),
        pltpu.VMEM_SHARED((N_ACC, D), jnp.float32),
        pltpu.SemaphoreType.DMA,
        pltpu.SemaphoreType.DMA,
        pltpu.SemaphoreType.DMA,
        pltpu.SemaphoreType.DMA,
        pltpu.SemaphoreType.DMA,
    ],
)


def _ea_sweep_body(ea_hbm, dst_hbm, z_hbm, out_ea, dst_v, eb0, eb1, acc,
                   zsem, sem0, sem1):
    c = lax.axis_index("c")
    s = lax.axis_index("s")
    wid = c * NS + s

    zbase = s * ZERO_PT
    pltpu.async_copy(z_hbm, acc.at[pl.ds(zbase, ZERO_PT)], zsem)
    pltpu.make_async_copy(z_hbm, acc.at[pl.ds(zbase, ZERO_PT)], zsem).wait()

    plsc.subcore_barrier()

    def chunk(ci, carry):
        pltpu.sync_copy(dst_hbm.at[wid, ci], dst_v)
        base = ci * CH
        pltpu.async_copy(ea_hbm.at[wid, base], eb0, sem0)
        pltpu.async_copy(ea_hbm.at[wid, base + 1], eb1, sem1)

        def step(eb, sem, j):
            pltpu.make_async_copy(ea_hbm.at[wid, base + j], eb, sem).wait()
            pltpu.sync_copy(eb, acc.at[dst_v.at[j]], add=True)

            @pl.when(j + 2 < CH)
            def _():
                pltpu.async_copy(ea_hbm.at[wid, base + j + 2], eb, sem)

        def pair(i, c2):
            step(eb0, sem0, 2 * i)
            step(eb1, sem1, 2 * i + 1)
            return c2

        lax.fori_loop(0, CH // 2, pair, 0)
        return carry

    lax.fori_loop(0, NCH, chunk, 0)

    plsc.subcore_barrier()

    pltpu.sync_copy(acc.at[pl.ds(zbase, ZERO_PT)],
                    out_ea.at[c, pl.ds(zbase, ZERO_PT)])


_sweep_ea = pl.kernel(
    _ea_sweep_body,
    out_type=(jax.ShapeDtypeStruct((NC, N_ACC, D), jnp.float32),),
    mesh=plsc.VectorSubcoreMesh(core_axis_name="c", subcore_axis_name="s"),
    scratch_types=[
        pltpu.VMEM((CH, K), jnp.int32),
        pltpu.VMEM((K, D), jnp.float32),
        pltpu.VMEM((K, D), jnp.float32),
        pltpu.VMEM_SHARED((N_ACC, D), jnp.float32),
        pltpu.SemaphoreType.DMA,
        pltpu.SemaphoreType.DMA,
        pltpu.SemaphoreType.DMA,
    ],
)


_RB = 1000
_NRB = N // _RB


def _combine1_body(px0, px1, pe0, pe1, h, We, Ws, b, oh, oea):
    ea = pe0[0][:, :DE] + pe1[0][:, :DE]
    r = (px0[0] + px1[0]
         + jnp.dot(ea, We[...], preferred_element_type=jnp.float32)
         + jnp.dot(h[...], Ws[...], preferred_element_type=jnp.float32)
         + b[...])
    oh[...] = jnp.maximum(r, 0.0)
    oea[...] = ea


def _combine2_body(px0, px1, ea, h, We, Ws, b, oh):
    r = (px0[0] + px1[0]
         + jnp.dot(ea[...], We[...], preferred_element_type=jnp.float32)
         + jnp.dot(h[...], Ws[...], preferred_element_type=jnp.float32)
         + b[...])
    oh[...] = jnp.maximum(r, 0.0)


def _part_spec(core, cols):
    return pl.BlockSpec((1, _RB, cols), lambda i, c=core: (c, i, 0))


def _row_spec(cols):
    return pl.BlockSpec((_RB, cols), lambda i: (i, 0))


def _full_spec(rows, cols):
    return pl.BlockSpec((rows, cols), lambda i: (0, 0))


_combine1 = pl.pallas_call(
    _combine1_body,
    grid=(_NRB,),
    in_specs=[
        _part_spec(0, D), _part_spec(1, D),
        _part_spec(0, D), _part_spec(1, D),
        _row_spec(D),
        _full_spec(DE, D), _full_spec(D, D), _full_spec(1, D),
    ],
    out_specs=[_row_spec(D), _row_spec(DE)],
    out_shape=[
        jax.ShapeDtypeStruct((N, D), jnp.float32),
        jax.ShapeDtypeStruct((N, DE), jnp.float32),
    ],
)

_combine2 = pl.pallas_call(
    _combine2_body,
    grid=(_NRB,),
    in_specs=[
        _part_spec(0, D), _part_spec(1, D),
        _row_spec(DE),
        _row_spec(D),
        _full_spec(DE, D), _full_spec(D, D), _full_spec(1, D),
    ],
    out_specs=_row_spec(D),
    out_shape=jax.ShapeDtypeStruct((N, D), jnp.float32),
)

_KB = 1280
_NKB = (DIM * D) // _KB


def _fc_body(g, W1, b1, W2, b2, o, acc):
    k = pl.program_id(0)

    @pl.when(k == 0)
    def _():
        acc[...] = jnp.zeros_like(acc)

    acc[...] += jnp.dot(g[...], W1[...], preferred_element_type=jnp.float32)

    @pl.when(k == _NKB - 1)
    def _():
        t = jnp.maximum(acc[...] + b1[...], 0.0)
        o[...] = (jnp.dot(t, W2[...], preferred_element_type=jnp.float32)
                  + b2[...])


_fc = pl.pallas_call(
    _fc_body,
    grid=(_NKB,),
    in_specs=[
        pl.BlockSpec((G, _KB), lambda k: (0, k)),
        pl.BlockSpec((_KB, H1), lambda k: (k, 0)),
        pl.BlockSpec((1, H1), lambda k: (0, 0)),
        pl.BlockSpec((H1, OUT), lambda k: (0, 0)),
        pl.BlockSpec((1, OUT), lambda k: (0, 0)),
    ],
    out_specs=pl.BlockSpec((G, OUT), lambda k: (0, 0)),
    out_shape=jax.ShapeDtypeStruct((G, OUT), jnp.float32),
    scratch_shapes=[pltpu.VMEM((G, H1), jnp.float32)],
)


def kernel(x, edge_index, edge_attr, We1, Ws1, b1, We2, Ws2, b2,
           W_fc1, b_fc1, W_fc2, b_fc2):
    pad = E_PAD - E
    src = jnp.concatenate(
        [edge_index[0].astype(jnp.int32), jnp.zeros((pad,), jnp.int32)])
    dst = jnp.concatenate(
        [edge_index[1].astype(jnp.int32), jnp.full((pad,), N, jnp.int32)])
    src_r = src.reshape(NW, NCH, CH, K)
    dst_r = dst.reshape(NW, NCH, CH, K)
    ea_r = jnp.pad(edge_attr, ((0, pad), (0, D - DE))).reshape(NW, NB, K, D)

    z_d = jnp.zeros((ZERO_PT, D), jnp.float32)

    (pea,) = _sweep_ea(ea_r, dst_r, z_d)
    (px,) = _sweep(x, src_r, dst_r, z_d)
    h1, ea = _combine1(px, px, pea, pea, x, We1, Ws1, b1.reshape(1, D))
    (ph,) = _sweep(h1, src_r, dst_r, z_d)
    h2 = _combine2(ph, ph, ea, h1, We2, Ws2, b2.reshape(1, D))
    return _fc(h2.reshape(G, DIM * D), W_fc1, b_fc1.reshape(1, H1),
               W_fc2, b_fc2.reshape(1, OUT))


# speedup vs baseline: 1.0362x; 1.0362x over previous
import functools

import jax
import jax.numpy as jnp
from jax import lax
from jax.experimental import pallas as pl
from jax.experimental.pallas import tpu as pltpu
from jax.experimental.pallas import tpu_sc as plsc

N = 10000
E = 320000
D = 128
DE = 16
DIM = 100
G = N // DIM
H1 = 512
OUT = 128

NC, NS = 2, 16
NW = NC * NS
K = 128
NB = 80
NCH = 4
CH = NB // NCH
E_PAD = NW * NB * K
N_ACC = 10240
ZERO_PT = N_ACC // NS


def _gather_sweep_body(h_hbm, src_hbm, dst_hbm, z_hbm, out_x,
                       src_v, dst_v, rb0, rb1, acc, zsem, sem0, sem1):
    c = lax.axis_index("c")
    s = lax.axis_index("s")
    wid = c * NS + s

    zbase = s * ZERO_PT
    pltpu.async_copy(z_hbm, acc.at[pl.ds(zbase, ZERO_PT)], zsem)
    pltpu.make_async_copy(z_hbm, acc.at[pl.ds(zbase, ZERO_PT)], zsem).wait()

    plsc.subcore_barrier()

    def chunk(ci, carry):
        pltpu.sync_copy(src_hbm.at[wid, ci], src_v)
        pltpu.sync_copy(dst_hbm.at[wid, ci], dst_v)
        pltpu.async_copy(h_hbm.at[src_v.at[0]], rb0, sem0)
        pltpu.async_copy(h_hbm.at[src_v.at[1]], rb1, sem1)

        def step(rb, sem, j):
            pltpu.make_async_copy(h_hbm.at[src_v.at[j]], rb, sem).wait()
            pltpu.sync_copy(rb, acc.at[dst_v.at[j]], add=True)

            @pl.when(j + 2 < CH)
            def _():
                pltpu.async_copy(h_hbm.at[src_v.at[j + 2]], rb, sem)

        def pair(i, c2):
            step(rb0, sem0, 2 * i)
            step(rb1, sem1, 2 * i + 1)
            return c2

        lax.fori_loop(0, CH // 2, pair, 0)
        return carry

    lax.fori_loop(0, NCH, chunk, 0)

    plsc.subcore_barrier()

    pltpu.sync_copy(acc.at[pl.ds(zbase, ZERO_PT)],
                    out_x.at[c, pl.ds(zbase, ZERO_PT)])


_sweep = pl.kernel(
    _gather_sweep_body,
    out_type=(jax.ShapeDtypeStruct((NC, N_ACC, D), jnp.float32),),
    mesh=plsc.VectorSubcoreMesh(core_axis_name="c", subcore_axis_name="s"),
    scratch_types=[
        pltpu.VMEM((CH, K), jnp.int32),
        pltpu.VMEM((CH, K), jnp.int32),
        pltpu.VMEM((K, D), jnp.float32),
        pltpu.VMEM((K, D), jnp.float32),
        pltpu.VMEM_SHARED((N_ACC, D), jnp.float32),
        pltpu.SemaphoreType.DMA,
        pltpu.SemaphoreType.DMA,
        pltpu.SemaphoreType.DMA,
    ],
)


def _ea_sweep_body(ea_hbm, dst_hbm, z_hbm, out_ea, dst_v, eb0, eb1, acc,
                   zsem, sem0, sem1):
    c = lax.axis_index("c")
    s = lax.axis_index("s")
    wid = c * NS + s

    zbase = s * ZERO_PT
    pltpu.async_copy(z_hbm, acc.at[pl.ds(zbase, ZERO_PT)], zsem)
    pltpu.make_async_copy(z_hbm, acc.at[pl.ds(zbase, ZERO_PT)], zsem).wait()

    plsc.subcore_barrier()

    def chunk(ci, carry):
        pltpu.sync_copy(dst_hbm.at[wid, ci], dst_v)
        base = ci * CH
        pltpu.async_copy(ea_hbm.at[wid, base], eb0, sem0)
        pltpu.async_copy(ea_hbm.at[wid, base + 1], eb1, sem1)

        def step(eb, sem, j):
            pltpu.make_async_copy(ea_hbm.at[wid, base + j], eb, sem).wait()
            pltpu.sync_copy(eb, acc.at[dst_v.at[j]], add=True)

            @pl.when(j + 2 < CH)
            def _():
                pltpu.async_copy(ea_hbm.at[wid, base + j + 2], eb, sem)

        def pair(i, c2):
            step(eb0, sem0, 2 * i)
            step(eb1, sem1, 2 * i + 1)
            return c2

        lax.fori_loop(0, CH // 2, pair, 0)
        return carry

    lax.fori_loop(0, NCH, chunk, 0)

    plsc.subcore_barrier()

    pltpu.sync_copy(acc.at[pl.ds(zbase, ZERO_PT)],
                    out_ea.at[c, pl.ds(zbase, ZERO_PT)])


_sweep_ea = pl.kernel(
    _ea_sweep_body,
    out_type=(jax.ShapeDtypeStruct((NC, N_ACC, D), jnp.float32),),
    mesh=plsc.VectorSubcoreMesh(core_axis_name="c", subcore_axis_name="s"),
    scratch_types=[
        pltpu.VMEM((CH, K), jnp.int32),
        pltpu.VMEM((K, D), jnp.float32),
        pltpu.VMEM((K, D), jnp.float32),
        pltpu.VMEM_SHARED((N_ACC, D), jnp.float32),
        pltpu.SemaphoreType.DMA,
        pltpu.SemaphoreType.DMA,
        pltpu.SemaphoreType.DMA,
    ],
)


_RB = 1000
_NRB = N // _RB


def _combine1_body(px0, px1, pe0, pe1, h, We, Ws, b, oh, oea):
    ea = pe0[0][:, :DE] + pe1[0][:, :DE]
    r = (px0[0] + px1[0]
         + jnp.dot(ea, We[...], preferred_element_type=jnp.float32)
         + jnp.dot(h[...], Ws[...], preferred_element_type=jnp.float32)
         + b[...])
    oh[...] = jnp.maximum(r, 0.0)
    oea[...] = ea


def _combine2_body(px0, px1, ea, h, We, Ws, b, oh):
    r = (px0[0] + px1[0]
         + jnp.dot(ea[...], We[...], preferred_element_type=jnp.float32)
         + jnp.dot(h[...], Ws[...], preferred_element_type=jnp.float32)
         + b[...])
    oh[...] = jnp.maximum(r, 0.0)


def _part_spec(core, cols):
    return pl.BlockSpec((1, _RB, cols), lambda i, c=core: (c, i, 0))


def _row_spec(cols):
    return pl.BlockSpec((_RB, cols), lambda i: (i, 0))


def _full_spec(rows, cols):
    return pl.BlockSpec((rows, cols), lambda i: (0, 0))


_combine1 = pl.pallas_call(
    _combine1_body,
    grid=(_NRB,),
    in_specs=[
        _part_spec(0, D), _part_spec(1, D),
        _part_spec(0, D), _part_spec(1, D),
        _row_spec(D),
        _full_spec(DE, D), _full_spec(D, D), _full_spec(1, D),
    ],
    out_specs=[_row_spec(D), _row_spec(DE)],
    out_shape=[
        jax.ShapeDtypeStruct((N, D), jnp.float32),
        jax.ShapeDtypeStruct((N, DE), jnp.float32),
    ],
)

_combine2 = pl.pallas_call(
    _combine2_body,
    grid=(_NRB,),
    in_specs=[
        _part_spec(0, D), _part_spec(1, D),
        _row_spec(DE),
        _row_spec(D),
        _full_spec(DE, D), _full_spec(D, D), _full_spec(1, D),
    ],
    out_specs=_row_spec(D),
    out_shape=jax.ShapeDtypeStruct((N, D), jnp.float32),
)

_KB = 1280
_NKB = (DIM * D) // _KB


def _fc_body(g, W1, b1, W2, b2, o, acc):
    k = pl.program_id(0)

    @pl.when(k == 0)
    def _():
        acc[...] = jnp.zeros_like(acc)

    acc[...] += jnp.dot(g[...], W1[...], preferred_element_type=jnp.float32)

    @pl.when(k == _NKB - 1)
    def _():
        t = jnp.maximum(acc[...] + b1[...], 0.0)
        o[...] = (jnp.dot(t, W2[...], preferred_element_type=jnp.float32)
                  + b2[...])


_fc = pl.pallas_call(
    _fc_body,
    grid=(_NKB,),
    in_specs=[
        pl.BlockSpec((G, _KB), lambda k: (0, k)),
        pl.BlockSpec((_KB, H1), lambda k: (k, 0)),
        pl.BlockSpec((1, H1), lambda k: (0, 0)),
        pl.BlockSpec((H1, OUT), lambda k: (0, 0)),
        pl.BlockSpec((1, OUT), lambda k: (0, 0)),
    ],
    out_specs=pl.BlockSpec((G, OUT), lambda k: (0, 0)),
    out_shape=jax.ShapeDtypeStruct((G, OUT), jnp.float32),
    scratch_shapes=[pltpu.VMEM((G, H1), jnp.float32)],
)


def kernel(x, edge_index, edge_attr, We1, Ws1, b1, We2, Ws2, b2,
           W_fc1, b_fc1, W_fc2, b_fc2):
    pad = E_PAD - E
    src = jnp.concatenate(
        [edge_index[0].astype(jnp.int32), jnp.zeros((pad,), jnp.int32)])
    dst = jnp.concatenate(
        [edge_index[1].astype(jnp.int32), jnp.full((pad,), N, jnp.int32)])
    src_r = src.reshape(NW, NCH, CH, K)
    dst_r = dst.reshape(NW, NCH, CH, K)
    ea_r = jnp.pad(edge_attr, ((0, pad), (0, D - DE))).reshape(NW, NB, K, D)

    z_d = jnp.zeros((ZERO_PT, D), jnp.float32)

    (pea,) = _sweep_ea(ea_r, dst_r, z_d)
    (px,) = _sweep(x, src_r, dst_r, z_d)
    h1, ea = _combine1(px, px, pea, pea, x, We1, Ws1, b1.reshape(1, D))
    (ph,) = _sweep(h1, src_r, dst_r, z_d)
    h2 = _combine2(ph, ph, ea, h1, We2, Ws2, b2.reshape(1, D))
    return _fc(h2.reshape(G, DIM * D), W_fc1, b_fc1.reshape(1, H1),
               W_fc2, b_fc2.reshape(1, OUT))
